# baseline (speedup 1.0000x reference)
import functools

import jax
import jax.numpy as jnp
from jax.experimental import pallas as pl
from jax.experimental.pallas import tpu as pltpu

_NUM_QUERIES = 5


def _ranksel_body(srow_ref, scol_ref, idx_ref, val_ref, rank_ref, *, ns, k, ch):
    n = srow_ref.shape[-1]
    srow = srow_ref[0]
    jio = jax.lax.broadcasted_iota(jnp.int32, (1, n), 1)
    trow = jnp.where(jio < ns, srow, -jnp.inf)

    def rank_chunk(c, _):
        base = c * ch
        sc = scol_ref[0, pl.ds(base, ch), :]
        iio = base + jax.lax.broadcasted_iota(jnp.int32, (ch, 1), 0)
        beat = (trow > sc) | ((trow == sc) & (jio < iio))
        rank_ref[pl.ds(base, ch), :] = jnp.sum(
            jnp.where(beat, 1.0, 0.0), axis=1, keepdims=True)
        return 0

    jax.lax.fori_loop(0, n // ch, rank_chunk, 0, unroll=False)

    r = rank_ref[...]
    icol = jax.lax.broadcasted_iota(jnp.int32, (n, 1), 0)
    is_sup = icol < ns
    keep = jnp.logical_or(jnp.logical_not(is_sup), r < k)
    pos = jnp.where(is_sup, r, (k + icol - ns).astype(jnp.float32))
    pos = jnp.where(keep, pos, jnp.float32(2 * n))
    scol = scol_ref[0]
    icolf = icol.astype(jnp.float32)

    def out_chunk(c, _):
        base = c * ch
        prow = (base + jax.lax.broadcasted_iota(jnp.int32, (1, ch), 1)).astype(
            jnp.float32)
        m = pos == prow
        idxv = jnp.sum(jnp.where(m, icolf, 0.0), axis=0, keepdims=True)
        valv = jnp.sum(jnp.where(m, scol, 0.0), axis=0, keepdims=True)
        idx_ref[0, :, pl.ds(base, ch)] = idxv.astype(jnp.int32)
        val_ref[0, :, pl.ds(base, ch)] = valv
        return 0

    jax.lax.fori_loop(0, n // ch, out_chunk, 0, unroll=False)


def _agather_body(idxc_ref, idxr_ref, a_ref, out_ref):
    n = a_ref.shape[-1]
    idxc = idxc_ref[0]
    ncols = jax.lax.broadcasted_iota(jnp.int32, (1, n), 1)
    pm = jnp.where(idxc == ncols, 1.0, 0.0)
    rows = jnp.dot(pm, a_ref[0], preferred_element_type=jnp.float32)
    idxr = idxr_ref[0]
    nrows = jax.lax.broadcasted_iota(jnp.int32, (n, 1), 0)
    qm = jnp.where(nrows == idxr, 1.0, 0.0)
    out_ref[0] = jnp.dot(rows, qm, preferred_element_type=jnp.float32)


def _xgather_body(idxc_ref, valc_ref, x_ref, out_ref):
    n = x_ref.shape[-2]
    idxc = idxc_ref[0]
    valc = valc_ref[0]
    ncols = jax.lax.broadcasted_iota(jnp.int32, (1, n), 1)
    pm = jnp.where(idxc == ncols, valc, 0.0)
    out_ref[0] = jnp.dot(pm, x_ref[0], preferred_element_type=jnp.float32)


def _ceil_to(x, m):
    return (x + m - 1) // m * m


def kernel(A, X, W, b):
    B, N, D = X.shape
    ns = N - _NUM_QUERIES
    k = ns // 2
    p_out = k + _NUM_QUERIES
    ch = 256 if N % 256 == 0 else N
    rb = 128
    pp = _ceil_to(p_out, rb)

    scores = jax.vmap(
        lambda Xi: jax.nn.sigmoid(((Xi @ W.T + b)[:, 0]) / 100.0))(X)

    s_row = scores[:, None, :]
    s_col = scores[:, :, None]

    idx_full, val_full = pl.pallas_call(
        functools.partial(_ranksel_body, ns=ns, k=k, ch=ch),
        grid=(B,),
        in_specs=[
            pl.BlockSpec((1, 1, N), lambda bi: (bi, 0, 0)),
            pl.BlockSpec((1, N, 1), lambda bi: (bi, 0, 0)),
        ],
        out_specs=[
            pl.BlockSpec((1, 1, N), lambda bi: (bi, 0, 0)),
            pl.BlockSpec((1, 1, N), lambda bi: (bi, 0, 0)),
        ],
        out_shape=[
            jax.ShapeDtypeStruct((B, 1, N), jnp.int32),
            jax.ShapeDtypeStruct((B, 1, N), jnp.float32),
        ],
        scratch_shapes=[pltpu.VMEM((N, 1), jnp.float32)],
    )(s_row, s_col)

    idx = idx_full[:, 0, :p_out]
    vals = val_full[:, 0, :p_out]

    idx_pad = jnp.pad(idx, ((0, 0), (0, pp - p_out)), constant_values=-1)
    vals_pad = jnp.pad(vals, ((0, 0), (0, pp - p_out)))
    idx_col = idx_pad[:, :, None]
    idx_row = idx_pad[:, None, :]
    vals_col = vals_pad[:, :, None]

    new_a_pad = pl.pallas_call(
        _agather_body,
        grid=(B, pp // rb),
        in_specs=[
            pl.BlockSpec((1, rb, 1), lambda bi, ri: (bi, ri, 0)),
            pl.BlockSpec((1, 1, pp), lambda bi, ri: (bi, 0, 0)),
            pl.BlockSpec((1, N, N), lambda bi, ri: (bi, 0, 0)),
        ],
        out_specs=pl.BlockSpec((1, rb, pp), lambda bi, ri: (bi, ri, 0)),
        out_shape=jax.ShapeDtypeStruct((B, pp, pp), jnp.float32),
    )(idx_col, idx_row, A)

    new_x_pad = pl.pallas_call(
        _xgather_body,
        grid=(B, pp // rb),
        in_specs=[
            pl.BlockSpec((1, rb, 1), lambda bi, ri: (bi, ri, 0)),
            pl.BlockSpec((1, rb, 1), lambda bi, ri: (bi, ri, 0)),
            pl.BlockSpec((1, N, D), lambda bi, ri: (bi, 0, 0)),
        ],
        out_specs=pl.BlockSpec((1, rb, D), lambda bi, ri: (bi, ri, 0)),
        out_shape=jax.ShapeDtypeStruct((B, pp, D), jnp.float32),
    )(idx_col, vals_col, X)

    return new_a_pad[:, :p_out, :p_out], new_x_pad[:, :p_out, :], idx


# speedup vs baseline: 1.2965x; 1.2965x over previous
import functools

import jax
import jax.numpy as jnp
from jax.experimental import pallas as pl
from jax.experimental.pallas import tpu as pltpu

_NUM_QUERIES = 5


def _ranksel_body(srow_ref, scol_ref, idx_ref, val_ref, rank_ref, *, ns, k, ch):
    n = srow_ref.shape[-1]
    srow = srow_ref[0]
    jio = jax.lax.broadcasted_iota(jnp.int32, (1, n), 1)
    trow = jnp.where(jio < ns, srow, -jnp.inf)

    def rank_chunk(c, _):
        base = c * ch
        sc = scol_ref[0, pl.ds(base, ch), :]
        iio = base + jax.lax.broadcasted_iota(jnp.int32, (ch, 1), 0)
        beat = (trow > sc) | ((trow == sc) & (jio < iio))
        rank_ref[pl.ds(base, ch), :] = jnp.sum(
            jnp.where(beat, 1.0, 0.0), axis=1, keepdims=True)
        return 0

    jax.lax.fori_loop(0, n // ch, rank_chunk, 0, unroll=False)

    r = rank_ref[...]
    icol = jax.lax.broadcasted_iota(jnp.int32, (n, 1), 0)
    is_sup = icol < ns
    keep = jnp.logical_or(jnp.logical_not(is_sup), r < k)
    pos = jnp.where(is_sup, r, (k + icol - ns).astype(jnp.float32))
    pos = jnp.where(keep, pos, jnp.float32(2 * n))
    scol = scol_ref[0]
    icolf = icol.astype(jnp.float32)

    def out_chunk(c, _):
        base = c * ch
        prow = (base + jax.lax.broadcasted_iota(jnp.int32, (1, ch), 1)).astype(
            jnp.float32)
        m = pos == prow
        idxv = jnp.sum(jnp.where(m, icolf, 0.0), axis=0, keepdims=True)
        valv = jnp.sum(jnp.where(m, scol, 0.0), axis=0, keepdims=True)
        idx_ref[0, :, pl.ds(base, ch)] = idxv.astype(jnp.int32)
        val_ref[0, :, pl.ds(base, ch)] = valv
        return 0

    jax.lax.fori_loop(0, n // ch, out_chunk, 0, unroll=False)


def _agather_body(idxc_ref, idxr_ref, a_ref, out_ref):
    n = a_ref.shape[-1]
    p_out = out_ref.shape[-1]
    idxc = idxc_ref[0]
    ncols = jax.lax.broadcasted_iota(jnp.int32, (1, n), 1)
    pm = jnp.where(idxc == ncols, 1.0, 0.0)
    rows = jnp.dot(pm, a_ref[0], preferred_element_type=jnp.float32)
    idxr = idxr_ref[0, :, :p_out]
    nrows = jax.lax.broadcasted_iota(jnp.int32, (n, 1), 0)
    qm = jnp.where(nrows == idxr, 1.0, 0.0)
    out_ref[0] = jnp.dot(rows, qm, preferred_element_type=jnp.float32)


def _xgather_body(idxc_ref, valc_ref, x_ref, out_ref):
    n = x_ref.shape[-2]
    idxc = idxc_ref[0]
    valc = valc_ref[0]
    ncols = jax.lax.broadcasted_iota(jnp.int32, (1, n), 1)
    pm = jnp.where(idxc == ncols, valc, 0.0)
    out_ref[0] = jnp.dot(pm, x_ref[0], preferred_element_type=jnp.float32)


def _ceil_to(x, m):
    return (x + m - 1) // m * m


def kernel(A, X, W, b):
    B, N, D = X.shape
    ns = N - _NUM_QUERIES
    k = ns // 2
    p_out = k + _NUM_QUERIES
    ch = 256 if N % 256 == 0 else N
    rb = 128
    pp = _ceil_to(p_out, rb)

    scores = jax.vmap(
        lambda Xi: jax.nn.sigmoid(((Xi @ W.T + b)[:, 0]) / 100.0))(X)

    s_row = scores[:, None, :]
    s_col = scores[:, :, None]

    idx_full, val_full = pl.pallas_call(
        functools.partial(_ranksel_body, ns=ns, k=k, ch=ch),
        grid=(B,),
        in_specs=[
            pl.BlockSpec((1, 1, N), lambda bi: (bi, 0, 0)),
            pl.BlockSpec((1, N, 1), lambda bi: (bi, 0, 0)),
        ],
        out_specs=[
            pl.BlockSpec((1, 1, N), lambda bi: (bi, 0, 0)),
            pl.BlockSpec((1, 1, N), lambda bi: (bi, 0, 0)),
        ],
        out_shape=[
            jax.ShapeDtypeStruct((B, 1, N), jnp.int32),
            jax.ShapeDtypeStruct((B, 1, N), jnp.float32),
        ],
        scratch_shapes=[pltpu.VMEM((N, 1), jnp.float32)],
    )(s_row, s_col)

    idx = idx_full[:, 0, :p_out]
    vals = val_full[:, 0, :p_out]

    idx_pad = jnp.pad(idx, ((0, 0), (0, pp - p_out)), constant_values=-1)
    vals_pad = jnp.pad(vals, ((0, 0), (0, pp - p_out)))
    idx_col = idx_pad[:, :, None]
    idx_row = idx_pad[:, None, :]
    vals_col = vals_pad[:, :, None]

    new_a = pl.pallas_call(
        _agather_body,
        grid=(B, pp // rb),
        in_specs=[
            pl.BlockSpec((1, rb, 1), lambda bi, ri: (bi, ri, 0)),
            pl.BlockSpec((1, 1, pp), lambda bi, ri: (bi, 0, 0)),
            pl.BlockSpec((1, N, N), lambda bi, ri: (bi, 0, 0)),
        ],
        out_specs=pl.BlockSpec((1, rb, p_out), lambda bi, ri: (bi, ri, 0)),
        out_shape=jax.ShapeDtypeStruct((B, p_out, p_out), jnp.float32),
    )(idx_col, idx_row, A)

    new_x = pl.pallas_call(
        _xgather_body,
        grid=(B, pp // rb),
        in_specs=[
            pl.BlockSpec((1, rb, 1), lambda bi, ri: (bi, ri, 0)),
            pl.BlockSpec((1, rb, 1), lambda bi, ri: (bi, ri, 0)),
            pl.BlockSpec((1, N, D), lambda bi, ri: (bi, 0, 0)),
        ],
        out_specs=pl.BlockSpec((1, rb, D), lambda bi, ri: (bi, ri, 0)),
        out_shape=jax.ShapeDtypeStruct((B, p_out, D), jnp.float32),
    )(idx_col, vals_col, X)

    return new_a, new_x, idx
